# baseline (speedup 1.0000x reference)
import functools

import jax
import jax.numpy as jnp
from jax import lax
from jax.experimental import pallas as pl
from jax.experimental.pallas import tpu as pltpu
from jax.experimental.pallas import tpu_sc as plsc

NUM_EMBEDDINGS = 1000000
EMBED_DIM = 64
N_TOKENS = 819200

_NC = 2
_NS = 16
_NW = _NC * _NS

_CHUNK = 128
_B_PER_W = N_TOKENS // _NW
_N_CHUNKS = _B_PER_W // _CHUNK

_mesh = plsc.VectorSubcoreMesh(core_axis_name="c", subcore_axis_name="s")


@functools.partial(
    pl.kernel,
    out_type=jax.ShapeDtypeStruct((_NW, _N_CHUNKS, _CHUNK, EMBED_DIM), jnp.float32),
    mesh=_mesh,
    scratch_types=[
        pltpu.VMEM((_N_CHUNKS, _CHUNK), jnp.int32),
        pltpu.VMEM((_CHUNK, EMBED_DIM), jnp.float32),
        pltpu.SemaphoreType.DMA,
    ],
    compiler_params=pltpu.CompilerParams(use_tc_tiling_on_sc=False),
)
def _gather_kernel(tokens_hbm, table_hbm, out_hbm, idx_v, rows_v, gsem):
    wid = lax.axis_index("s") * _NC + lax.axis_index("c")
    pltpu.sync_copy(tokens_hbm.at[wid], idx_v)

    def step(j, carry):
        pltpu.async_copy(table_hbm.at[idx_v.at[j]], rows_v, gsem).wait()
        pltpu.sync_copy(rows_v, out_hbm.at[wid, j])
        return carry

    lax.fori_loop(0, _N_CHUNKS, step, 0)


def kernel(tokens, table):
    tok3 = tokens.reshape(_NW, _N_CHUNKS, _CHUNK)
    out = _gather_kernel(tok3, table)
    return out.reshape(N_TOKENS, EMBED_DIM)


# speedup vs baseline: 1.1129x; 1.1129x over previous
import functools

import jax
import jax.numpy as jnp
from jax import lax
from jax.experimental import pallas as pl
from jax.experimental.pallas import tpu as pltpu
from jax.experimental.pallas import tpu_sc as plsc

NUM_EMBEDDINGS = 1000000
EMBED_DIM = 64
N_TOKENS = 819200

_NC = 2
_NS = 16
_NW = _NC * _NS

_CHUNK = 128
_B_PER_W = N_TOKENS // _NW
_N_CHUNKS = _B_PER_W // _CHUNK
_K = 4
_N_GROUPS = _N_CHUNKS // _K

_mesh = plsc.VectorSubcoreMesh(core_axis_name="c", subcore_axis_name="s")


@functools.partial(
    pl.kernel,
    out_type=jax.ShapeDtypeStruct((_NW, _N_CHUNKS, _CHUNK, EMBED_DIM), jnp.float32),
    mesh=_mesh,
    scratch_types=[
        pltpu.VMEM((_N_CHUNKS, _CHUNK), jnp.int32),
        pltpu.VMEM((2, _K, _CHUNK, EMBED_DIM), jnp.float32),
        pltpu.SemaphoreType.DMA,
        pltpu.SemaphoreType.DMA,
    ],
    compiler_params=pltpu.CompilerParams(use_tc_tiling_on_sc=False),
)
def _gather_kernel(tokens_hbm, table_hbm, out_hbm, idx_v, rows_v, gsem, wsem):
    wid = lax.axis_index("s") * _NC + lax.axis_index("c")
    pltpu.sync_copy(tokens_hbm.at[wid], idx_v)

    def fire_gathers(g, half):
        for b in range(_K):
            pltpu.async_copy(
                table_hbm.at[idx_v.at[g * _K + b]], rows_v.at[half, b], gsem)

    def drain_gathers(g, half):
        for b in range(_K):
            pltpu.make_async_copy(
                table_hbm.at[idx_v.at[g * _K + b]], rows_v.at[half, b], gsem).wait()

    def fire_writes(g, half):
        for b in range(_K):
            pltpu.async_copy(
                rows_v.at[half, b], out_hbm.at[wid, g * _K + b], wsem)

    def drain_writes(g, half):
        for b in range(_K):
            pltpu.make_async_copy(
                rows_v.at[half, b], out_hbm.at[wid, g * _K + b], wsem).wait()

    fire_gathers(0, 0)
    drain_gathers(0, 0)
    fire_gathers(1, 1)
    fire_writes(0, 0)

    def body(g, carry):
        half = g % 2
        other = 1 - half
        drain_gathers(g, half)
        drain_writes(g - 1, other)
        fire_gathers(g + 1, other)
        fire_writes(g, half)
        return carry

    lax.fori_loop(1, _N_GROUPS - 1, body, 0)

    g = _N_GROUPS - 1
    half = g % 2
    drain_gathers(g, half)
    drain_writes(g - 1, 1 - half)
    fire_writes(g, half)
    drain_writes(g, half)


def kernel(tokens, table):
    tok3 = tokens.reshape(_NW, _N_CHUNKS, _CHUNK)
    out = _gather_kernel(tok3, table)
    return out.reshape(N_TOKENS, EMBED_DIM)


# speedup vs baseline: 1.1138x; 1.0009x over previous
import functools

import jax
import jax.numpy as jnp
from jax import lax
from jax.experimental import pallas as pl
from jax.experimental.pallas import tpu as pltpu
from jax.experimental.pallas import tpu_sc as plsc

NUM_EMBEDDINGS = 1000000
EMBED_DIM = 64
N_TOKENS = 819200

_NC = 2
_NS = 16
_NW = _NC * _NS

_CHUNK = 128
_B_PER_W = N_TOKENS // _NW
_N_CHUNKS = _B_PER_W // _CHUNK
_K = 4
_N_GROUPS = _N_CHUNKS // _K

_mesh = plsc.VectorSubcoreMesh(core_axis_name="c", subcore_axis_name="s")


@functools.partial(
    pl.kernel,
    out_type=jax.ShapeDtypeStruct((N_TOKENS, EMBED_DIM), jnp.float32),
    mesh=_mesh,
    scratch_types=[
        pltpu.VMEM((_B_PER_W,), jnp.int32),
        pltpu.VMEM((2, _K, _CHUNK, EMBED_DIM), jnp.float32),
        pltpu.SemaphoreType.DMA,
        pltpu.SemaphoreType.DMA,
    ],
    compiler_params=pltpu.CompilerParams(use_tc_tiling_on_sc=False),
)
def _gather_kernel(tokens_hbm, table_hbm, out_hbm, idx_v, rows_v, gsem, wsem):
    wid = lax.axis_index("s") * _NC + lax.axis_index("c")
    base = wid * _B_PER_W
    pltpu.sync_copy(tokens_hbm.at[pl.ds(base, _B_PER_W)], idx_v)

    def fire_gathers(g, half):
        for b in range(_K):
            pltpu.async_copy(
                table_hbm.at[idx_v.at[pl.ds((g * _K + b) * _CHUNK, _CHUNK)]],
                rows_v.at[half, b], gsem)

    def drain_gathers(g, half):
        for b in range(_K):
            pltpu.make_async_copy(
                table_hbm.at[idx_v.at[pl.ds((g * _K + b) * _CHUNK, _CHUNK)]],
                rows_v.at[half, b], gsem).wait()

    def fire_writes(g, half):
        for b in range(_K):
            pltpu.async_copy(
                rows_v.at[half, b],
                out_hbm.at[pl.ds(base + (g * _K + b) * _CHUNK, _CHUNK)], wsem)

    def drain_writes(g, half):
        for b in range(_K):
            pltpu.make_async_copy(
                rows_v.at[half, b],
                out_hbm.at[pl.ds(base + (g * _K + b) * _CHUNK, _CHUNK)], wsem).wait()

    fire_gathers(0, 0)
    drain_gathers(0, 0)
    fire_gathers(1, 1)
    fire_writes(0, 0)

    def body(g, carry):
        half = g % 2
        other = 1 - half
        drain_gathers(g, half)
        drain_writes(g - 1, other)
        fire_gathers(g + 1, other)
        fire_writes(g, half)
        return carry

    lax.fori_loop(1, _N_GROUPS - 1, body, 0)

    g = _N_GROUPS - 1
    half = g % 2
    drain_gathers(g, half)
    drain_writes(g - 1, 1 - half)
    fire_writes(g, half)
    drain_writes(g, half)


def kernel(tokens, table):
    return _gather_kernel(tokens, table)
